# baseline (speedup 1.0000x reference)
import functools

import jax
import jax.numpy as jnp
from jax import lax
from jax.experimental import pallas as pl
from jax.experimental.pallas import tpu as pltpu
from jax.experimental.pallas import tpu_sc as plsc

_B, _N, _STRIDE, _K = 2, 8192, 4, 32
_S = _N // _STRIDE
_CLP, _CIN, _COUT, _CMID = 16, 128, 128, 32
_EPS = 1e-5
_HIGH = lax.Precision.HIGHEST

_NT1 = 512


def _post_body(f_ref, w_ref, z_ref, st_ref):
    b = pl.program_id(0)
    n = pl.program_id(1)
    ftile = f_ref[0]
    w = w_ref[...]
    z = lax.dot_general(ftile, w, (((0,), (1,)), ((), ())),
                        preferred_element_type=jnp.float32)
    z_ref[0] = z

    @pl.when((b == 0) & (n == 0))
    def _():
        st_ref[...] = jnp.zeros_like(st_ref)

    st_ref[0:1, :] += jnp.sum(z, axis=0)[None, :]
    st_ref[1:2, :] += jnp.sum(z * z, axis=0)[None, :]


def _postconv(f, w_post):
    return pl.pallas_call(
        _post_body,
        grid=(_B, _N // _NT1),
        in_specs=[
            pl.BlockSpec((1, _CIN, _NT1), lambda b, n: (b, 0, n)),
            pl.BlockSpec((_COUT, _CIN), lambda b, n: (0, 0)),
        ],
        out_specs=[
            pl.BlockSpec((1, _NT1, _COUT), lambda b, n: (b, n, 0)),
            pl.BlockSpec((8, 128), lambda b, n: (0, 0)),
        ],
        out_shape=[
            jax.ShapeDtypeStruct((_B, _N, _COUT), jnp.float32),
            jax.ShapeDtypeStruct((8, 128), jnp.float32),
        ],
    )(f, w_post)


_ST2 = 128
_CN2 = 512
_NCH = _N // _CN2


def _bq_body(q_ref, p_ref, tri_ref, g_ref, cnt_ref):
    b = pl.program_id(0)
    c = pl.program_id(2)

    @pl.when(c == 0)
    def _():
        cnt_ref[...] = jnp.zeros_like(cnt_ref)
        g_ref[...] = jnp.zeros_like(g_ref)

    prev = cnt_ref[...][:, 0:1]

    @pl.when(jnp.min(prev) < float(_K))
    def _():
        q = q_ref[0]
        pc = p_ref[0]
        dx = q[:, 0:1] - pc[0:1, :]
        dy = q[:, 1:2] - pc[1:2, :]
        dz = q[:, 2:3] - pc[2:3, :]
        d2 = (dx * dx + dy * dy) + dz * dz
        mask = d2 < (0.15 ** 2)
        mf = jnp.where(mask, 1.0, 0.0)
        rowhits = jnp.sum(mf, axis=1, keepdims=True)
        cs = lax.dot_general(mf.astype(jnp.bfloat16), tri_ref[...],
                             (((1,), (0,)), ((), ())),
                             preferred_element_type=jnp.float32)
        ci = cs + prev
        jglob = (lax.broadcasted_iota(jnp.int32, (_ST2, _CN2), 1).astype(
            jnp.float32) + (c * _CN2 + b * _N).astype(jnp.float32))
        active = prev < float(_K)
        kstart = jnp.min(jnp.where(active, prev, 1e9))
        kend = jnp.max(jnp.where(
            active, jnp.minimum(prev + rowhits, float(_K)), 0.0))
        for k in range(_K):
            @pl.when((kstart <= float(k)) & (float(k) < kend))
            def _(k=k):
                sel = mask & (ci == float(k + 1))
                contrib = jnp.sum(jnp.where(sel, jglob, 0.0), axis=1,
                                  keepdims=True)
                g_ref[0, :, k:k + 1] += contrib
        cnt_ref[...] = cnt_ref[...] + rowhits

    @pl.when(c == _NCH - 1)
    def _():
        g = g_ref[0]
        cnt = cnt_ref[...][:, 0:1]
        kio2 = lax.broadcasted_iota(jnp.int32, (_ST2, _K), 1).astype(
            jnp.float32)
        g_ref[0] = jnp.where(kio2 < cnt, g, g[:, 0:1])


def _ball_query(new_p, p2, tri):
    return pl.pallas_call(
        _bq_body,
        grid=(_B, _S // _ST2, _NCH),
        in_specs=[
            pl.BlockSpec((1, _ST2, 3), lambda b, s, c: (b, s, 0)),
            pl.BlockSpec((1, 3, _CN2), lambda b, s, c: (b, 0, c)),
            pl.BlockSpec((_CN2, _CN2), lambda b, s, c: (0, 0)),
        ],
        out_specs=pl.BlockSpec((1, _ST2, _K), lambda b, s, c: (b, s, 0)),
        out_shape=jax.ShapeDtypeStruct((_B, _S, _K), jnp.float32),
        scratch_shapes=[pltpu.VMEM((_ST2, 128), jnp.float32)],
    )(new_p, p2, tri)


def _sc_center_gather(table_m, idx_flat):
    info = plsc.get_sparse_core_info()
    nw = info.num_cores * info.num_subcores
    nidx = idx_flat.shape[0]
    per_w = nidx // nw
    mesh = plsc.VectorSubcoreMesh(core_axis_name="c", subcore_axis_name="s")

    @functools.partial(
        pl.kernel, mesh=mesh,
        out_type=jax.ShapeDtypeStruct((nidx, 128), jnp.float32),
        scratch_types=[
            pltpu.VMEM((per_w,), jnp.int32),
            pltpu.VMEM((per_w, 128), jnp.float32),
            pltpu.SemaphoreType.DMA,
        ],
    )
    def k(tab_hbm, idx_hbm, out, idx_v, buf, sem):
        wid = lax.axis_index("s") * info.num_cores + lax.axis_index("c")
        base = wid * per_w
        pltpu.sync_copy(idx_hbm.at[pl.ds(base, per_w)], idx_v)
        pltpu.async_copy(tab_hbm.at[idx_v], buf, sem).wait()
        pltpu.sync_copy(buf, out.at[pl.ds(base, per_w)])

    return k(table_m, idx_flat)


def _sc_neighbor_gather(table_m, table_z, gidx_flat):
    info = plsc.get_sparse_core_info()
    nw = info.num_cores * info.num_subcores
    nidx = gidx_flat.shape[0]
    chunk = 128
    per_w = nidx // nw
    nloop = per_w // chunk
    mesh = plsc.VectorSubcoreMesh(core_axis_name="c", subcore_axis_name="s")

    @functools.partial(
        pl.kernel, mesh=mesh,
        out_type=[
            jax.ShapeDtypeStruct((nidx, 128), jnp.float32),
            jax.ShapeDtypeStruct((nidx, 128), jnp.float32),
        ],
        scratch_types=[
            pltpu.VMEM((chunk,), jnp.int32),
            pltpu.VMEM((chunk, 128), jnp.float32),
            pltpu.VMEM((chunk, 128), jnp.float32),
            pltpu.SemaphoreType.DMA,
            pltpu.SemaphoreType.DMA,
        ],
    )
    def k(m_hbm, z_hbm, idx_hbm, out_lpj, out_fj, idx_v, bl, bz, s1, s2):
        wid = lax.axis_index("s") * info.num_cores + lax.axis_index("c")
        base = wid * per_w

        def body(i, carry):
            off = base + i * chunk
            pltpu.sync_copy(idx_hbm.at[pl.ds(off, chunk)], idx_v)
            c1 = pltpu.async_copy(m_hbm.at[idx_v], bl, s1)
            c2 = pltpu.async_copy(z_hbm.at[idx_v], bz, s2)
            c1.wait()
            c2.wait()
            pltpu.sync_copy(bl, out_lpj.at[pl.ds(off, chunk)])
            pltpu.sync_copy(bz, out_fj.at[pl.ds(off, chunk)])
            return carry

        lax.fori_loop(0, nloop, body, 0)

    return k(table_m, table_z, gidx_flat)


_MT = 4096
_NM = _B * _S * _K // _MT


def _p1_body(lpj_ref, li_ref, s_ref, c_ref):
    g = pl.program_id(0)
    d = lpj_ref[...] - li_ref[...]

    @pl.when(g == 0)
    def _():
        s_ref[...] = jnp.zeros_like(s_ref)
        c_ref[...] = jnp.zeros_like(c_ref)

    s_ref[...] += jnp.sum(d, axis=0)[None, :]
    c_ref[...] += lax.dot_general(d, d, (((0,), (0,)), ((), ())),
                                  preferred_element_type=jnp.float32)


def _pass1(lpj2, lirep):
    return pl.pallas_call(
        _p1_body,
        grid=(_NM,),
        in_specs=[
            pl.BlockSpec((_MT, _CLP), lambda g: (g, 0)),
            pl.BlockSpec((_MT, _CLP), lambda g: (g, 0)),
        ],
        out_specs=[
            pl.BlockSpec((1, _CLP), lambda g: (0, 0)),
            pl.BlockSpec((_CLP, _CLP), lambda g: (0, 0)),
        ],
        out_shape=[
            jax.ShapeDtypeStruct((1, _CLP), jnp.float32),
            jax.ShapeDtypeStruct((_CLP, _CLP), jnp.float32),
        ],
    )(lpj2, lirep)


def _p2_body(lpj_ref, li_ref, w1_ref, a1_ref, s_ref, c_ref):
    g = pl.program_id(0)
    d = lpj_ref[...] - li_ref[...]
    h1 = lax.dot_general(d, w1_ref[...], (((1,), (1,)), ((), ())),
                         preferred_element_type=jnp.float32)
    h1n = jnp.maximum(h1 * a1_ref[0:1, :] + a1_ref[1:2, :], 0.0)

    @pl.when(g == 0)
    def _():
        s_ref[...] = jnp.zeros_like(s_ref)
        c_ref[...] = jnp.zeros_like(c_ref)

    s_ref[...] += jnp.sum(h1n, axis=0)[None, :]
    c_ref[...] += lax.dot_general(h1n, h1n, (((0,), (0,)), ((), ())),
                                  preferred_element_type=jnp.float32)


def _pass2(lpj2, lirep, w1, aff1):
    return pl.pallas_call(
        _p2_body,
        grid=(_NM,),
        in_specs=[
            pl.BlockSpec((_MT, _CLP), lambda g: (g, 0)),
            pl.BlockSpec((_MT, _CLP), lambda g: (g, 0)),
            pl.BlockSpec((_CMID, _CLP), lambda g: (0, 0)),
            pl.BlockSpec((2, _CMID), lambda g: (0, 0)),
        ],
        out_specs=[
            pl.BlockSpec((1, _CMID), lambda g: (0, 0)),
            pl.BlockSpec((_CMID, _CMID), lambda g: (0, 0)),
        ],
        out_shape=[
            jax.ShapeDtypeStruct((1, _CMID), jnp.float32),
            jax.ShapeDtypeStruct((_CMID, _CMID), jnp.float32),
        ],
    )(lpj2, lirep, w1, aff1)


def _p3_body(lpj_ref, li_ref, fj_ref, w1_ref, a1_ref, w2_ref, a2_ref,
             ap_ref, o_ref):
    d = lpj_ref[...] - li_ref[...]
    h1 = lax.dot_general(d, w1_ref[...], (((1,), (1,)), ((), ())),
                         preferred_element_type=jnp.float32)
    h1n = jnp.maximum(h1 * a1_ref[0:1, :] + a1_ref[1:2, :], 0.0)
    h2 = lax.dot_general(h1n, w2_ref[...], (((1,), (1,)), ((), ())),
                         preferred_element_type=jnp.float32)
    h = jnp.maximum(h2 * a2_ref[0:1, :] + a2_ref[1:2, :], 0.0)
    fj = jnp.maximum(fj_ref[...] * ap_ref[0:1, :] + ap_ref[1:2, :], 0.0)
    t = (fj + h).reshape(_MT // _K, _K, _COUT)
    o_ref[...] = jnp.max(t, axis=1)


def _pass3(lpj2, lirep, fj2, w1, aff1, w2, aff2, affp):
    return pl.pallas_call(
        _p3_body,
        grid=(_NM,),
        in_specs=[
            pl.BlockSpec((_MT, _CLP), lambda g: (g, 0)),
            pl.BlockSpec((_MT, _CLP), lambda g: (g, 0)),
            pl.BlockSpec((_MT, _COUT), lambda g: (g, 0)),
            pl.BlockSpec((_CMID, _CLP), lambda g: (0, 0)),
            pl.BlockSpec((2, _CMID), lambda g: (0, 0)),
            pl.BlockSpec((_COUT, _CMID), lambda g: (0, 0)),
            pl.BlockSpec((2, _COUT), lambda g: (0, 0)),
            pl.BlockSpec((2, _COUT), lambda g: (0, 0)),
        ],
        out_specs=pl.BlockSpec((_MT // _K, _COUT), lambda g: (g, 0)),
        out_shape=jax.ShapeDtypeStruct((_B * _S, _COUT), jnp.float32),
    )(lpj2, lirep, fj2, w1, aff1, w2, aff2, affp)


def kernel(p, lp, f, W_post, g_post, b_post, W_loc1, g1, b1, W_loc2, g2, b2):
    skey = jax.random.key(42)
    idx = jax.vmap(lambda k: jax.random.permutation(k, _N)[:_S])(
        jax.random.split(skey, _B))

    lp_t = jnp.transpose(lp, (0, 2, 1))
    table_m = jnp.concatenate(
        [lp_t, p, jnp.zeros((_B, _N, 128 - _CLP - 3), jnp.float32)], axis=-1
    ).reshape(_B * _N, 128)
    p2 = jnp.transpose(p, (0, 2, 1))

    offs = (jnp.arange(_B, dtype=jnp.int32) * _N)[:, None]
    idx_flat = (idx.astype(jnp.int32) + offs).reshape(-1)

    centers = _sc_center_gather(table_m, idx_flat).reshape(_B, _S, 128)
    new_p = centers[:, :, _CLP:_CLP + 3]
    lpi_t = centers[:, :, :_CLP]
    lpi = jnp.transpose(lpi_t, (0, 2, 1))

    z_t, st = _postconv(f, W_post)
    cntn = float(_B * _N)
    mean_p = st[0, :] / cntn
    var_p = st[1, :] / cntn - mean_p * mean_p
    a_p = g_post / jnp.sqrt(var_p + _EPS)
    c_p = b_post - mean_p * a_p
    affp = jnp.stack([a_p, c_p])

    ar = jnp.arange(_CN2, dtype=jnp.int32)
    tri = (ar[:, None] <= ar[None, :]).astype(jnp.bfloat16)
    gidx_f = _ball_query(new_p, p2, tri)
    gidx_flat = gidx_f.astype(jnp.int32).reshape(-1)

    lpj_rows, fj_rows = _sc_neighbor_gather(
        table_m, z_t.reshape(_B * _N, _COUT), gidx_flat)
    bm = _B * _S * _K
    lpj2 = lpj_rows.reshape(bm, 8, _CLP)[:, 0, :]
    lirep = jnp.broadcast_to(
        lpi_t.reshape(_B * _S, 1, _CLP), (_B * _S, _K, _CLP)).reshape(bm, _CLP)

    msamp = float(bm)
    s1, c1m = _pass1(lpj2, lirep)
    mu_d = s1[0] / msamp
    cov_d = c1m / msamp - jnp.outer(mu_d, mu_d)
    m1 = W_loc1 @ mu_d
    v1 = jnp.sum((W_loc1 @ cov_d) * W_loc1, axis=1)
    a1 = g1 / jnp.sqrt(v1 + _EPS)
    aff1 = jnp.stack([a1, b1 - m1 * a1])

    s2, c2m = _pass2(lpj2, lirep, W_loc1, aff1)
    mu_h = s2[0] / msamp
    cov_h = c2m / msamp - jnp.outer(mu_h, mu_h)
    m2 = W_loc2 @ mu_h
    v2 = jnp.sum((W_loc2 @ cov_h) * W_loc2, axis=1)
    a2 = g2 / jnp.sqrt(v2 + _EPS)
    aff2 = jnp.stack([a2, b2 - m2 * a2])

    fout_t = _pass3(lpj2, lirep, fj_rows, W_loc1, aff1, W_loc2, aff2, affp)
    fout = jnp.transpose(fout_t.reshape(_B, _S, _COUT), (0, 2, 1))
    return (new_p, lpi, fout)


# speedup vs baseline: 1.3735x; 1.3735x over previous
import functools

import jax
import jax.numpy as jnp
from jax import lax
from jax.experimental import pallas as pl
from jax.experimental.pallas import tpu as pltpu
from jax.experimental.pallas import tpu_sc as plsc

_B, _N, _STRIDE, _K = 2, 8192, 4, 32
_S = _N // _STRIDE
_CLP, _CIN, _COUT, _CMID = 16, 128, 128, 32
_EPS = 1e-5
_HIGH = lax.Precision.HIGHEST

_NT1 = 512


def _post_body(f_ref, w_ref, z_ref, st_ref):
    b = pl.program_id(0)
    n = pl.program_id(1)
    ftile = f_ref[0]
    w = w_ref[...]
    z = lax.dot_general(ftile, w, (((0,), (1,)), ((), ())),
                        preferred_element_type=jnp.float32)
    z_ref[0] = z

    @pl.when((b == 0) & (n == 0))
    def _():
        st_ref[...] = jnp.zeros_like(st_ref)

    st_ref[0:1, :] += jnp.sum(z, axis=0)[None, :]
    st_ref[1:2, :] += jnp.sum(z * z, axis=0)[None, :]


def _postconv(f, w_post):
    return pl.pallas_call(
        _post_body,
        grid=(_B, _N // _NT1),
        in_specs=[
            pl.BlockSpec((1, _CIN, _NT1), lambda b, n: (b, 0, n)),
            pl.BlockSpec((_COUT, _CIN), lambda b, n: (0, 0)),
        ],
        out_specs=[
            pl.BlockSpec((1, _NT1, _COUT), lambda b, n: (b, n, 0)),
            pl.BlockSpec((8, 128), lambda b, n: (0, 0)),
        ],
        out_shape=[
            jax.ShapeDtypeStruct((_B, _N, _COUT), jnp.float32),
            jax.ShapeDtypeStruct((8, 128), jnp.float32),
        ],
    )(f, w_post)


_ST2 = 128
_CN2 = 512
_NCH = _N // _CN2


def _bq_body(q_ref, p_ref, tri_ref, g_ref, cnt_ref):
    b = pl.program_id(0)
    c = pl.program_id(2)

    @pl.when(c == 0)
    def _():
        cnt_ref[...] = jnp.zeros_like(cnt_ref)
        g_ref[...] = jnp.zeros_like(g_ref)

    prev = cnt_ref[...][:, 0:1]

    @pl.when(jnp.min(prev) < float(_K))
    def _():
        q = q_ref[0]
        pc = p_ref[0]
        dx = q[:, 0:1] - pc[0:1, :]
        dy = q[:, 1:2] - pc[1:2, :]
        dz = q[:, 2:3] - pc[2:3, :]
        d2 = (dx * dx + dy * dy) + dz * dz
        mask = d2 < (0.15 ** 2)
        mf = jnp.where(mask, 1.0, 0.0)
        rowhits = jnp.sum(mf, axis=1, keepdims=True)
        cs = lax.dot_general(mf.astype(jnp.bfloat16), tri_ref[...],
                             (((1,), (0,)), ((), ())),
                             preferred_element_type=jnp.float32)
        ci = cs + prev
        cim = jnp.where(mask, ci, 0.0)
        jglob = (lax.broadcasted_iota(jnp.int32, (_ST2, _CN2), 1).astype(
            jnp.float32) + (c * _CN2 + b * _N).astype(jnp.float32))
        acc = jnp.zeros((_ST2, _K), jnp.float32)
        kio = lax.broadcasted_iota(jnp.int32, (1, _K), 1)
        for k in range(_K):
            contrib = jnp.sum(jnp.where(cim == float(k + 1), jglob, 0.0),
                              axis=1, keepdims=True)
            acc = acc + contrib * jnp.where(kio == k, 1.0, 0.0)
        g_ref[0] += acc
        cnt_ref[...] = cnt_ref[...] + rowhits

    @pl.when(c == _NCH - 1)
    def _():
        g = g_ref[0]
        cnt = cnt_ref[...][:, 0:1]
        kio2 = lax.broadcasted_iota(jnp.int32, (_ST2, _K), 1).astype(
            jnp.float32)
        g_ref[0] = jnp.where(kio2 < cnt, g, g[:, 0:1])


def _ball_query(new_p, p2, tri):
    return pl.pallas_call(
        _bq_body,
        grid=(_B, _S // _ST2, _NCH),
        in_specs=[
            pl.BlockSpec((1, _ST2, 3), lambda b, s, c: (b, s, 0)),
            pl.BlockSpec((1, 3, _CN2), lambda b, s, c: (b, 0, c)),
            pl.BlockSpec((_CN2, _CN2), lambda b, s, c: (0, 0)),
        ],
        out_specs=pl.BlockSpec((1, _ST2, _K), lambda b, s, c: (b, s, 0)),
        out_shape=jax.ShapeDtypeStruct((_B, _S, _K), jnp.float32),
        scratch_shapes=[pltpu.VMEM((_ST2, 128), jnp.float32)],
    )(new_p, p2, tri)


def _sc_center_gather(table_m, idx_flat):
    info = plsc.get_sparse_core_info()
    nw = info.num_cores * info.num_subcores
    nidx = idx_flat.shape[0]
    per_w = nidx // nw
    mesh = plsc.VectorSubcoreMesh(core_axis_name="c", subcore_axis_name="s")

    @functools.partial(
        pl.kernel, mesh=mesh,
        out_type=jax.ShapeDtypeStruct((nidx, 128), jnp.float32),
        scratch_types=[
            pltpu.VMEM((per_w,), jnp.int32),
            pltpu.VMEM((per_w, 128), jnp.float32),
            pltpu.SemaphoreType.DMA,
        ],
    )
    def k(tab_hbm, idx_hbm, out, idx_v, buf, sem):
        wid = lax.axis_index("s") * info.num_cores + lax.axis_index("c")
        base = wid * per_w
        pltpu.sync_copy(idx_hbm.at[pl.ds(base, per_w)], idx_v)
        pltpu.async_copy(tab_hbm.at[idx_v], buf, sem).wait()
        pltpu.sync_copy(buf, out.at[pl.ds(base, per_w)])

    return k(table_m, idx_flat)


def _sc_neighbor_gather(table_m, table_z, gidx_flat):
    info = plsc.get_sparse_core_info()
    nw = info.num_cores * info.num_subcores
    nidx = gidx_flat.shape[0]
    chunk = 128
    per_w = nidx // nw
    nloop = per_w // chunk
    mesh = plsc.VectorSubcoreMesh(core_axis_name="c", subcore_axis_name="s")

    @functools.partial(
        pl.kernel, mesh=mesh,
        out_type=[
            jax.ShapeDtypeStruct((nidx, 128), jnp.float32),
            jax.ShapeDtypeStruct((nidx, 128), jnp.float32),
        ],
        scratch_types=[
            pltpu.VMEM((chunk,), jnp.int32),
            pltpu.VMEM((chunk, 128), jnp.float32),
            pltpu.VMEM((chunk, 128), jnp.float32),
            pltpu.SemaphoreType.DMA,
            pltpu.SemaphoreType.DMA,
        ],
    )
    def k(m_hbm, z_hbm, idx_hbm, out_lpj, out_fj, idx_v, bl, bz, s1, s2):
        wid = lax.axis_index("s") * info.num_cores + lax.axis_index("c")
        base = wid * per_w

        def body(i, carry):
            off = base + i * chunk
            pltpu.sync_copy(idx_hbm.at[pl.ds(off, chunk)], idx_v)
            c1 = pltpu.async_copy(m_hbm.at[idx_v], bl, s1)
            c2 = pltpu.async_copy(z_hbm.at[idx_v], bz, s2)
            c1.wait()
            c2.wait()
            pltpu.sync_copy(bl, out_lpj.at[pl.ds(off, chunk)])
            pltpu.sync_copy(bz, out_fj.at[pl.ds(off, chunk)])
            return carry

        lax.fori_loop(0, nloop, body, 0)

    return k(table_m, table_z, gidx_flat)


_MT = 4096
_NM = _B * _S * _K // _MT


def _p1_body(lpj_ref, li_ref, s_ref, c_ref):
    g = pl.program_id(0)
    d = lpj_ref[...] - li_ref[...]

    @pl.when(g == 0)
    def _():
        s_ref[...] = jnp.zeros_like(s_ref)
        c_ref[...] = jnp.zeros_like(c_ref)

    s_ref[...] += jnp.sum(d, axis=0)[None, :]
    c_ref[...] += lax.dot_general(d, d, (((0,), (0,)), ((), ())),
                                  preferred_element_type=jnp.float32)


def _pass1(lpj2, lirep):
    return pl.pallas_call(
        _p1_body,
        grid=(_NM,),
        in_specs=[
            pl.BlockSpec((_MT, _CLP), lambda g: (g, 0)),
            pl.BlockSpec((_MT, _CLP), lambda g: (g, 0)),
        ],
        out_specs=[
            pl.BlockSpec((1, _CLP), lambda g: (0, 0)),
            pl.BlockSpec((_CLP, _CLP), lambda g: (0, 0)),
        ],
        out_shape=[
            jax.ShapeDtypeStruct((1, _CLP), jnp.float32),
            jax.ShapeDtypeStruct((_CLP, _CLP), jnp.float32),
        ],
    )(lpj2, lirep)


def _p2_body(lpj_ref, li_ref, w1_ref, a1_ref, s_ref, c_ref):
    g = pl.program_id(0)
    d = lpj_ref[...] - li_ref[...]
    h1 = lax.dot_general(d, w1_ref[...], (((1,), (1,)), ((), ())),
                         preferred_element_type=jnp.float32)
    h1n = jnp.maximum(h1 * a1_ref[0:1, :] + a1_ref[1:2, :], 0.0)

    @pl.when(g == 0)
    def _():
        s_ref[...] = jnp.zeros_like(s_ref)
        c_ref[...] = jnp.zeros_like(c_ref)

    s_ref[...] += jnp.sum(h1n, axis=0)[None, :]
    c_ref[...] += lax.dot_general(h1n, h1n, (((0,), (0,)), ((), ())),
                                  preferred_element_type=jnp.float32)


def _pass2(lpj2, lirep, w1, aff1):
    return pl.pallas_call(
        _p2_body,
        grid=(_NM,),
        in_specs=[
            pl.BlockSpec((_MT, _CLP), lambda g: (g, 0)),
            pl.BlockSpec((_MT, _CLP), lambda g: (g, 0)),
            pl.BlockSpec((_CMID, _CLP), lambda g: (0, 0)),
            pl.BlockSpec((2, _CMID), lambda g: (0, 0)),
        ],
        out_specs=[
            pl.BlockSpec((1, _CMID), lambda g: (0, 0)),
            pl.BlockSpec((_CMID, _CMID), lambda g: (0, 0)),
        ],
        out_shape=[
            jax.ShapeDtypeStruct((1, _CMID), jnp.float32),
            jax.ShapeDtypeStruct((_CMID, _CMID), jnp.float32),
        ],
    )(lpj2, lirep, w1, aff1)


def _p3_body(lpj_ref, li_ref, fj_ref, w1_ref, a1_ref, w2_ref, a2_ref,
             ap_ref, o_ref):
    d = lpj_ref[...] - li_ref[...]
    h1 = lax.dot_general(d, w1_ref[...], (((1,), (1,)), ((), ())),
                         preferred_element_type=jnp.float32)
    h1n = jnp.maximum(h1 * a1_ref[0:1, :] + a1_ref[1:2, :], 0.0)
    h2 = lax.dot_general(h1n, w2_ref[...], (((1,), (1,)), ((), ())),
                         preferred_element_type=jnp.float32)
    h = jnp.maximum(h2 * a2_ref[0:1, :] + a2_ref[1:2, :], 0.0)
    fj = jnp.maximum(fj_ref[...] * ap_ref[0:1, :] + ap_ref[1:2, :], 0.0)
    t = (fj + h).reshape(_MT // _K, _K, _COUT)
    o_ref[...] = jnp.max(t, axis=1)


def _pass3(lpj2, lirep, fj2, w1, aff1, w2, aff2, affp):
    return pl.pallas_call(
        _p3_body,
        grid=(_NM,),
        in_specs=[
            pl.BlockSpec((_MT, _CLP), lambda g: (g, 0)),
            pl.BlockSpec((_MT, _CLP), lambda g: (g, 0)),
            pl.BlockSpec((_MT, _COUT), lambda g: (g, 0)),
            pl.BlockSpec((_CMID, _CLP), lambda g: (0, 0)),
            pl.BlockSpec((2, _CMID), lambda g: (0, 0)),
            pl.BlockSpec((_COUT, _CMID), lambda g: (0, 0)),
            pl.BlockSpec((2, _COUT), lambda g: (0, 0)),
            pl.BlockSpec((2, _COUT), lambda g: (0, 0)),
        ],
        out_specs=pl.BlockSpec((_MT // _K, _COUT), lambda g: (g, 0)),
        out_shape=jax.ShapeDtypeStruct((_B * _S, _COUT), jnp.float32),
    )(lpj2, lirep, fj2, w1, aff1, w2, aff2, affp)


def kernel(p, lp, f, W_post, g_post, b_post, W_loc1, g1, b1, W_loc2, g2, b2):
    skey = jax.random.key(42)
    idx = jax.vmap(lambda k: jax.random.permutation(k, _N)[:_S])(
        jax.random.split(skey, _B))

    lp_t = jnp.transpose(lp, (0, 2, 1))
    table_m = jnp.concatenate(
        [lp_t, p, jnp.zeros((_B, _N, 128 - _CLP - 3), jnp.float32)], axis=-1
    ).reshape(_B * _N, 128)
    p2 = jnp.transpose(p, (0, 2, 1))

    offs = (jnp.arange(_B, dtype=jnp.int32) * _N)[:, None]
    idx_flat = (idx.astype(jnp.int32) + offs).reshape(-1)

    centers = _sc_center_gather(table_m, idx_flat).reshape(_B, _S, 128)
    new_p = centers[:, :, _CLP:_CLP + 3]
    lpi_t = centers[:, :, :_CLP]
    lpi = jnp.transpose(lpi_t, (0, 2, 1))

    z_t, st = _postconv(f, W_post)
    cntn = float(_B * _N)
    mean_p = st[0, :] / cntn
    var_p = st[1, :] / cntn - mean_p * mean_p
    a_p = g_post / jnp.sqrt(var_p + _EPS)
    c_p = b_post - mean_p * a_p
    affp = jnp.stack([a_p, c_p])

    ar = jnp.arange(_CN2, dtype=jnp.int32)
    tri = (ar[:, None] <= ar[None, :]).astype(jnp.bfloat16)
    gidx_f = _ball_query(new_p, p2, tri)
    gidx_flat = gidx_f.astype(jnp.int32).reshape(-1)

    lpj_rows, fj_rows = _sc_neighbor_gather(
        table_m, z_t.reshape(_B * _N, _COUT), gidx_flat)
    bm = _B * _S * _K
    lpj2 = lpj_rows.reshape(bm, 8, _CLP)[:, 0, :]
    lirep = jnp.broadcast_to(
        lpi_t.reshape(_B * _S, 1, _CLP), (_B * _S, _K, _CLP)).reshape(bm, _CLP)

    msamp = float(bm)
    s1, c1m = _pass1(lpj2, lirep)
    mu_d = s1[0] / msamp
    cov_d = c1m / msamp - jnp.outer(mu_d, mu_d)
    m1 = W_loc1 @ mu_d
    v1 = jnp.sum((W_loc1 @ cov_d) * W_loc1, axis=1)
    a1 = g1 / jnp.sqrt(v1 + _EPS)
    aff1 = jnp.stack([a1, b1 - m1 * a1])

    s2, c2m = _pass2(lpj2, lirep, W_loc1, aff1)
    mu_h = s2[0] / msamp
    cov_h = c2m / msamp - jnp.outer(mu_h, mu_h)
    m2 = W_loc2 @ mu_h
    v2 = jnp.sum((W_loc2 @ cov_h) * W_loc2, axis=1)
    a2 = g2 / jnp.sqrt(v2 + _EPS)
    aff2 = jnp.stack([a2, b2 - m2 * a2])

    fout_t = _pass3(lpj2, lirep, fj_rows, W_loc1, aff1, W_loc2, aff2, affp)
    fout = jnp.transpose(fout_t.reshape(_B, _S, _COUT), (0, 2, 1))
    return (new_p, lpi, fout)
